# baseline (speedup 1.0000x reference)
import jax
import jax.numpy as jnp
from jax import lax
from jax.experimental import pallas as pl
from jax.experimental.pallas import tpu as pltpu
from jax.experimental.pallas import tpu_sc as plsc

B = 16384
D = 32
NC = 2
NS = 16
NW = NC * NS
BPW = B // NW


def _body(v_ids, e_ids, t_ids, vw, ew, tw, vp, ep, tp,
          out_v, out_e, out_t,
          idx_bufs, w_bufs, p_bufs, gather_sems, store_sem):
    wid = lax.axis_index("s") * NC + lax.axis_index("c")
    base = wid * BPW

    tables = (
        (v_ids, vw, vp, out_v),
        (e_ids, ew, ep, out_e),
        (t_ids, tw, tp, out_t),
    )

    copies = []
    for k, (ids, w, p, _) in enumerate(tables):
        pltpu.sync_copy(ids.at[pl.ds(base, BPW)], idx_bufs[k])
        cw = pltpu.async_copy(w.at[idx_bufs[k]], w_bufs[k], gather_sems[k])
        cp = pltpu.async_copy(p.at[idx_bufs[k]], p_bufs[k], gather_sems[k])
        copies.append((cw, cp))

    store_copies = []
    for k, (_, _, _, out) in enumerate(tables):
        copies[k][0].wait()
        copies[k][1].wait()
        aw, ap = w_bufs[k], p_bufs[k]

        @plsc.parallel_loop(0, BPW, unroll=8)
        def _add_rows(i, aw=aw, ap=ap):
            aw[i, pl.ds(0, 16)] = aw[i, pl.ds(0, 16)] + ap[i, pl.ds(0, 16)]
            aw[i, pl.ds(16, 16)] = aw[i, pl.ds(16, 16)] + ap[i, pl.ds(16, 16)]

        store_copies.append(
            pltpu.async_copy(aw, out.at[pl.ds(base, BPW)], store_sem))

    for c in store_copies:
        c.wait()


@jax.jit
def _run(v_ids, e_ids, t_ids, vw, ew, tw, vp, ep, tp):
    mesh = plsc.VectorSubcoreMesh(
        core_axis_name="c", subcore_axis_name="s",
        num_cores=NC, num_subcores=NS)
    out_sds = jax.ShapeDtypeStruct((B, D), jnp.float32)
    kern = pl.kernel(
        _body,
        out_type=[out_sds, out_sds, out_sds],
        mesh=mesh,
        scratch_types=[
            [pltpu.VMEM((BPW,), jnp.int32) for _ in range(3)],
            [pltpu.VMEM((BPW, D), jnp.float32) for _ in range(3)],
            [pltpu.VMEM((BPW, D), jnp.float32) for _ in range(3)],
            [pltpu.SemaphoreType.DMA for _ in range(3)],
            pltpu.SemaphoreType.DMA,
        ],
        compiler_params=pltpu.CompilerParams(use_tc_tiling_on_sc=False),
    )
    return tuple(kern(v_ids, e_ids, t_ids, vw, ew, tw, vp, ep, tp))


def kernel(vertex_ids, edge_ids, triangle_ids,
           vertex_embed_w, edge_embed_w, triangle_embed_w,
           vertex_pos, edge_pos, triangle_pos):
    return _run(vertex_ids.astype(jnp.int32),
                edge_ids.astype(jnp.int32),
                triangle_ids.astype(jnp.int32),
                vertex_embed_w, edge_embed_w, triangle_embed_w,
                vertex_pos, edge_pos, triangle_pos)


# speedup vs baseline: 1.8546x; 1.8546x over previous
import jax
import jax.numpy as jnp
from jax import lax
from jax.experimental import pallas as pl
from jax.experimental.pallas import tpu as pltpu
from jax.experimental.pallas import tpu_sc as plsc

B = 16384
D = 32
NC = 2
NS = 16
NW = NC * NS
OUTR = B + NW
NV, NE, NT = 100000, 1000000, 1000000
FV, FE, FT = NV // 128, NE // 128, NT // 128


def _body(v_ids, e_ids, t_ids, vw, ew, tw, vp, ep, tp, tlv, tle, tlt,
          out_v, out_e, out_t,
          idx_all, hit_i, hit_b, pid_b, stw, stp, pi_b, pb_b, res,
          gsem, ssem):
    wid = lax.axis_index("s") * NC + lax.axis_index("c")
    base_dump = B + wid
    c16 = lax.iota(jnp.int32, 16)

    tables = (
        (v_ids, vw, vp, tlv, out_v, FV, 0, 4),
        (e_ids, ew, ep, tle, out_e, FE, 2, 2),
        (t_ids, tw, tp, tlt, out_t, FT, 2, 2),
    )

    for ids, w, p, tail_s, out, F, PL, NG in tables:
        lo = (wid * F) // NW
        hi = ((wid + 1) * F) // NW
        hi_sel = jnp.where(wid == NW - 1, jnp.int32(1 << 30), hi)

        pltpu.sync_copy(ids, idx_all)

        def scan(g, n, lo=lo, hi_sel=hi_sel):
            v = idx_all[pl.ds(g * 16, 16)]
            tc = v >> 7
            mask = (tc >= lo) & (tc < hi_sel)
            plsc.store_compressed(hit_i.at[pl.ds(n, 16)], v, mask=mask)
            plsc.store_compressed(hit_b.at[pl.ds(n, 16)], c16 + g * 16,
                                  mask=mask)
            return n + jnp.max(plsc.all_reduce_population_count(mask))

        n = lax.fori_loop(0, B // 16, scan, jnp.int32(0), unroll=2)
        hit_i[pl.ds(n, 16)] = jnp.full((16,), lo * 128, jnp.int32)
        hit_b[pl.ds(n, 16)] = jnp.full((16,), base_dump, jnp.int32)
        ngrp = n // 16 + 1

        def pidp(g, _, lo=lo, PL=PL):
            v = hit_i[pl.ds(g * 16, 16)]
            pid_b[pl.ds(g * 16, 16)] = ((v >> 7) - lo) >> PL
            return 0

        lax.fori_loop(0, ngrp, pidp, 0)

        np_ = ((hi - lo) + (1 << PL) - 1) >> PL

        def stage_start(p_, lo=lo, F=F, PL=PL):
            return jnp.minimum(lo + (p_ << PL), F - 4) * 128

        def issue(p_, buf, w=w, p=p):
            s = stage_start(p_)
            pltpu.async_copy(w.at[:, pl.ds(s, 512)], stw[buf], gsem)
            pltpu.async_copy(p.at[:, pl.ds(s, 512)], stp[buf], gsem)

        def wait_stage(buf, w=w):
            pltpu.make_async_copy(w.at[:, pl.ds(0, 512)], stw[buf],
                                  gsem).wait()
            pltpu.make_async_copy(w.at[:, pl.ds(0, 512)], stp[buf],
                                  gsem).wait()

        def wait_scatter(buf, out=out):
            pltpu.make_async_copy(res[buf], out.at[pb_b[buf]], ssem).wait()

        def piece(p_, buf, np_=np_, NG=NG, w=w, p=p, out=out):
            @pl.when(p_ < np_)
            def _():
                issue(jnp.minimum(p_ + 1, np_ - 1), 1 - buf)

                @pl.when(p_ >= 2)
                def _():
                    wait_scatter(buf)

                s = stage_start(p_)
                fill_i = jnp.broadcast_to(s, (16,))
                fill_b = jnp.full((16,), base_dump, jnp.int32)
                for g in range(4):
                    pi_b[buf][pl.ds(g * 16, 16)] = fill_i
                    pb_b[buf][pl.ds(g * 16, 16)] = fill_b

                def sel(g, m, p_=p_):
                    mask = pid_b[pl.ds(g * 16, 16)] == p_
                    plsc.store_compressed(
                        pi_b[buf].at[pl.ds(m, 16)],
                        hit_i[pl.ds(g * 16, 16)], mask=mask)
                    plsc.store_compressed(
                        pb_b[buf].at[pl.ds(m, 16)],
                        hit_b[pl.ds(g * 16, 16)], mask=mask)
                    return m + jnp.max(
                        plsc.all_reduce_population_count(mask))

                lax.fori_loop(0, ngrp, sel, jnp.int32(0))

                wait_stage(buf)

                def grp(g, _, s=s):
                    iv = pi_b[buf][pl.ds(g * 16, 16)]
                    col = iv - s
                    h16 = c16 + g * 16

                    def feat(c, _):
                        cc = jnp.broadcast_to(c, (16,))
                        wv = plsc.load_gather(stw[buf], [cc, col])
                        pv = plsc.load_gather(stp[buf], [cc, col])
                        plsc.store_scatter(res[buf], [h16, cc], wv + pv)
                        return 0

                    lax.fori_loop(0, D, feat, 0, unroll=4)
                    return 0

                lax.fori_loop(0, NG, grp, 0)
                pltpu.async_copy(res[buf], out.at[pb_b[buf]], ssem)

        issue(0, 0)

        def pair(q, _):
            piece(2 * q, 0)
            piece(2 * q + 1, 1)
            return 0

        lax.fori_loop(0, (np_ + 1) // 2, pair, 0)

        wait_stage(0)

        @pl.when(np_ >= 1)
        def _():
            wait_scatter(0)

        @pl.when(np_ >= 2)
        def _():
            wait_scatter(1)

        pltpu.sync_copy(tail_s, stw[0].at[:, pl.ds(0, 128)])
        fill_b = jnp.full((16,), base_dump, jnp.int32)
        tbase = jnp.int32(F * 128)
        fill_i = jnp.broadcast_to(tbase, (16,))
        for g in range(2):
            pi_b[0][pl.ds(g * 16, 16)] = fill_i
            pb_b[0][pl.ds(g * 16, 16)] = fill_b

        def tsel(g, m, tbase=tbase):
            v = hit_i[pl.ds(g * 16, 16)]
            mask = v >= tbase
            plsc.store_compressed(pi_b[0].at[pl.ds(m, 16)], v, mask=mask)
            plsc.store_compressed(pb_b[0].at[pl.ds(m, 16)],
                                  hit_b[pl.ds(g * 16, 16)], mask=mask)
            return m + jnp.max(plsc.all_reduce_population_count(mask))

        lax.fori_loop(0, ngrp, tsel, jnp.int32(0))

        def tgrp(g, _, tbase=tbase):
            iv = pi_b[0][pl.ds(g * 16, 16)]
            col = iv - tbase
            h16 = c16 + g * 16

            def feat(c, _):
                cc = jnp.broadcast_to(c, (16,))
                sv = plsc.load_gather(stw[0], [cc, col])
                plsc.store_scatter(res[0], [h16, cc], sv)
                return 0

            lax.fori_loop(0, D, feat, 0, unroll=4)
            return 0

        lax.fori_loop(0, 2, tgrp, 0)
        pltpu.async_copy(res[0], out.at[pb_b[0]], ssem)
        wait_scatter(0)


@jax.jit
def _run(v_ids, e_ids, t_ids, vw, ew, tw, vp, ep, tp):
    def tail_sum(w, p, F, N):
        t = (w[F * 128:] + p[F * 128:]).T
        return jnp.zeros((D, 128), jnp.float32).at[:, :N - F * 128].set(t)

    tlv = tail_sum(vw, vp, FV, NV)
    tle = tail_sum(ew, ep, FE, NE)
    tlt = tail_sum(tw, tp, FT, NT)

    mesh = plsc.VectorSubcoreMesh(
        core_axis_name="c", subcore_axis_name="s",
        num_cores=NC, num_subcores=NS)
    out_sds = jax.ShapeDtypeStruct((OUTR, 128), jnp.float32)
    kern = pl.kernel(
        _body,
        out_type=[out_sds, out_sds, out_sds],
        mesh=mesh,
        scratch_types=[
            pltpu.VMEM((B,), jnp.int32),
            pltpu.VMEM((1104,), jnp.int32),
            pltpu.VMEM((1104,), jnp.int32),
            pltpu.VMEM((1104,), jnp.int32),
            [pltpu.VMEM((D, 512), jnp.float32) for _ in range(2)],
            [pltpu.VMEM((D, 512), jnp.float32) for _ in range(2)],
            [pltpu.VMEM((64,), jnp.int32) for _ in range(2)],
            [pltpu.VMEM((64,), jnp.int32) for _ in range(2)],
            [pltpu.VMEM((64, 128), jnp.float32) for _ in range(2)],
            pltpu.SemaphoreType.DMA,
            pltpu.SemaphoreType.DMA,
        ],
        compiler_params=pltpu.CompilerParams(needs_layout_passes=False),
    )
    ov, oe, ot = kern(v_ids, e_ids, t_ids,
                      vw.T, ew.T, tw.T, vp.T, ep.T, tp.T, tlv, tle, tlt)
    return ov[:B, :D], oe[:B, :D], ot[:B, :D]


def kernel(vertex_ids, edge_ids, triangle_ids,
           vertex_embed_w, edge_embed_w, triangle_embed_w,
           vertex_pos, edge_pos, triangle_pos):
    return _run(vertex_ids.astype(jnp.int32),
                edge_ids.astype(jnp.int32),
                triangle_ids.astype(jnp.int32),
                vertex_embed_w, edge_embed_w, triangle_embed_w,
                vertex_pos, edge_pos, triangle_pos)


# speedup vs baseline: 1.9557x; 1.0545x over previous
import jax
import jax.numpy as jnp
from jax import lax
from jax.experimental import pallas as pl
from jax.experimental.pallas import tpu as pltpu
from jax.experimental.pallas import tpu_sc as plsc

B = 16384
D = 32
NC = 2
NS = 16
NW = NC * NS
OUTR = B + NW
NV, NE, NT = 100000, 1000000, 1000000
FV, FE, FT = NV // 128, NE // 128, NT // 128


def _body(v_ids, e_ids, t_ids, vw, ew, tw, vp, ep, tp, tlv, tle, tlt,
          out_v, out_e, out_t,
          idx_all, hit_i, hit_b, pid_b, stw, stp, pi_b, pb_b, res,
          gsem, ssem):
    wid = lax.axis_index("s") * NC + lax.axis_index("c")
    base_dump = B + wid
    c16 = lax.iota(jnp.int32, 16)

    tables = (
        (v_ids, vw, vp, tlv, out_v, FV, 0, 4, 128),
        (e_ids, ew, ep, tle, out_e, FE, 2, 2, 512),
        (t_ids, tw, tp, tlt, out_t, FT, 2, 2, 512),
    )

    for ids, w, p, tail_s, out, F, PL, NG, SW in tables:
        lo = (wid * F) // NW
        hi = ((wid + 1) * F) // NW
        hi_sel = jnp.where(wid == NW - 1, jnp.int32(1 << 30), hi)

        pltpu.sync_copy(ids, idx_all)

        def scan(g, n, lo=lo, hi_sel=hi_sel):
            v = idx_all[pl.ds(g * 16, 16)]
            tc = v >> 7
            mask = (tc >= lo) & (tc < hi_sel)
            plsc.store_compressed(hit_i.at[pl.ds(n, 16)], v, mask=mask)
            plsc.store_compressed(hit_b.at[pl.ds(n, 16)], c16 + g * 16,
                                  mask=mask)
            return n + plsc.all_reduce_population_count(mask)[0]

        n = lax.fori_loop(0, B // 16, scan, jnp.int32(0), unroll=2)
        hit_i[pl.ds(n, 16)] = jnp.full((16,), lo * 128, jnp.int32)
        hit_b[pl.ds(n, 16)] = jnp.full((16,), base_dump, jnp.int32)
        ngrp = n // 16 + 1

        def pidp(g, _, lo=lo, PL=PL):
            v = hit_i[pl.ds(g * 16, 16)]
            pid_b[pl.ds(g * 16, 16)] = ((v >> 7) - lo) >> PL
            return 0

        lax.fori_loop(0, ngrp, pidp, 0)

        np_ = ((hi - lo) + (1 << PL) - 1) >> PL

        def stage_start(p_, lo=lo, F=F, PL=PL, SW=SW):
            return jnp.minimum(lo + (p_ << PL), F - SW // 128) * 128

        def issue(p_, buf, w=w, p=p, SW=SW):
            s = stage_start(p_)
            pltpu.async_copy(w.at[:, pl.ds(s, SW)],
                             stw[buf].at[:, pl.ds(0, SW)], gsem)
            pltpu.async_copy(p.at[:, pl.ds(s, SW)],
                             stp[buf].at[:, pl.ds(0, SW)], gsem)

        def wait_stage(buf, w=w, SW=SW):
            pltpu.make_async_copy(w.at[:, pl.ds(0, SW)],
                                  stw[buf].at[:, pl.ds(0, SW)], gsem).wait()
            pltpu.make_async_copy(w.at[:, pl.ds(0, SW)],
                                  stp[buf].at[:, pl.ds(0, SW)], gsem).wait()

        def wait_scatter(buf, out=out):
            pltpu.make_async_copy(res[buf], out.at[pb_b[buf]], ssem).wait()

        def piece(p_, buf, np_=np_, NG=NG, w=w, p=p, out=out):
            @pl.when(p_ < np_)
            def _():
                issue(jnp.minimum(p_ + 1, np_ - 1), 1 - buf)

                @pl.when(p_ >= 2)
                def _():
                    wait_scatter(buf)

                s = stage_start(p_)
                fill_i = jnp.broadcast_to(s, (16,))
                fill_b = jnp.full((16,), base_dump, jnp.int32)
                for g in range(4):
                    pi_b[buf][pl.ds(g * 16, 16)] = fill_i
                    pb_b[buf][pl.ds(g * 16, 16)] = fill_b

                def sel(g, m, p_=p_):
                    mask = pid_b[pl.ds(g * 16, 16)] == p_
                    plsc.store_compressed(
                        pi_b[buf].at[pl.ds(m, 16)],
                        hit_i[pl.ds(g * 16, 16)], mask=mask)
                    plsc.store_compressed(
                        pb_b[buf].at[pl.ds(m, 16)],
                        hit_b[pl.ds(g * 16, 16)], mask=mask)
                    return m + plsc.all_reduce_population_count(mask)[0]

                lax.fori_loop(0, ngrp, sel, jnp.int32(0))

                wait_stage(buf)

                def grp(g, _, s=s):
                    iv = pi_b[buf][pl.ds(g * 16, 16)]
                    col = iv - s
                    h16 = c16 + g * 16

                    def feat(c, _):
                        cc = jnp.broadcast_to(c, (16,))
                        wv = plsc.load_gather(stw[buf], [cc, col])
                        pv = plsc.load_gather(stp[buf], [cc, col])
                        plsc.store_scatter(res[buf], [h16, cc], wv + pv)
                        return 0

                    lax.fori_loop(0, D, feat, 0, unroll=4)
                    return 0

                lax.fori_loop(0, NG, grp, 0)
                pltpu.async_copy(res[buf], out.at[pb_b[buf]], ssem)

        issue(0, 0)

        def pair(q, _):
            piece(2 * q, 0)
            piece(2 * q + 1, 1)
            return 0

        lax.fori_loop(0, (np_ + 1) // 2, pair, 0)

        wait_stage(0)

        @pl.when(np_ >= 1)
        def _():
            wait_scatter(0)

        @pl.when(np_ >= 2)
        def _():
            wait_scatter(1)

        pltpu.sync_copy(tail_s, stw[0].at[:, pl.ds(0, 128)])
        fill_b = jnp.full((16,), base_dump, jnp.int32)
        tbase = jnp.int32(F * 128)
        fill_i = jnp.broadcast_to(tbase, (16,))
        for g in range(2):
            pi_b[0][pl.ds(g * 16, 16)] = fill_i
            pb_b[0][pl.ds(g * 16, 16)] = fill_b

        def tsel(g, m, tbase=tbase):
            v = hit_i[pl.ds(g * 16, 16)]
            mask = v >= tbase
            plsc.store_compressed(pi_b[0].at[pl.ds(m, 16)], v, mask=mask)
            plsc.store_compressed(pb_b[0].at[pl.ds(m, 16)],
                                  hit_b[pl.ds(g * 16, 16)], mask=mask)
            return m + plsc.all_reduce_population_count(mask)[0]

        lax.fori_loop(0, ngrp, tsel, jnp.int32(0))

        def tgrp(g, _, tbase=tbase):
            iv = pi_b[0][pl.ds(g * 16, 16)]
            col = iv - tbase
            h16 = c16 + g * 16

            def feat(c, _):
                cc = jnp.broadcast_to(c, (16,))
                sv = plsc.load_gather(stw[0], [cc, col])
                plsc.store_scatter(res[0], [h16, cc], sv)
                return 0

            lax.fori_loop(0, D, feat, 0, unroll=4)
            return 0

        lax.fori_loop(0, 2, tgrp, 0)
        pltpu.async_copy(res[0], out.at[pb_b[0]], ssem)
        wait_scatter(0)


@jax.jit
def _run(v_ids, e_ids, t_ids, vw, ew, tw, vp, ep, tp):
    def tail_sum(w, p, F, N):
        t = (w[F * 128:] + p[F * 128:]).T
        return jnp.zeros((D, 128), jnp.float32).at[:, :N - F * 128].set(t)

    tlv = tail_sum(vw, vp, FV, NV)
    tle = tail_sum(ew, ep, FE, NE)
    tlt = tail_sum(tw, tp, FT, NT)

    mesh = plsc.VectorSubcoreMesh(
        core_axis_name="c", subcore_axis_name="s",
        num_cores=NC, num_subcores=NS)
    out_sds = jax.ShapeDtypeStruct((OUTR, 128), jnp.float32)
    kern = pl.kernel(
        _body,
        out_type=[out_sds, out_sds, out_sds],
        mesh=mesh,
        scratch_types=[
            pltpu.VMEM((B,), jnp.int32),
            pltpu.VMEM((1104,), jnp.int32),
            pltpu.VMEM((1104,), jnp.int32),
            pltpu.VMEM((1104,), jnp.int32),
            [pltpu.VMEM((D, 512), jnp.float32) for _ in range(2)],
            [pltpu.VMEM((D, 512), jnp.float32) for _ in range(2)],
            [pltpu.VMEM((64,), jnp.int32) for _ in range(2)],
            [pltpu.VMEM((64,), jnp.int32) for _ in range(2)],
            [pltpu.VMEM((64, 128), jnp.float32) for _ in range(2)],
            pltpu.SemaphoreType.DMA,
            pltpu.SemaphoreType.DMA,
        ],
        compiler_params=pltpu.CompilerParams(needs_layout_passes=False),
    )
    ov, oe, ot = kern(v_ids, e_ids, t_ids,
                      vw.T, ew.T, tw.T, vp.T, ep.T, tp.T, tlv, tle, tlt)
    return ov[:B, :D], oe[:B, :D], ot[:B, :D]


def kernel(vertex_ids, edge_ids, triangle_ids,
           vertex_embed_w, edge_embed_w, triangle_embed_w,
           vertex_pos, edge_pos, triangle_pos):
    return _run(vertex_ids.astype(jnp.int32),
                edge_ids.astype(jnp.int32),
                triangle_ids.astype(jnp.int32),
                vertex_embed_w, edge_embed_w, triangle_embed_w,
                vertex_pos, edge_pos, triangle_pos)
